# Initial kernel scaffold; baseline (speedup 1.0000x reference)
import jax
import jax.numpy as jnp
from jax.experimental import pallas as pl


def kernel(y_hat, y, W1, W2):
    raise NotImplementedError("write your pallas kernel here")


# speedup vs baseline: 1.2691x; 1.2691x over previous
import jax
import jax.numpy as jnp
from jax.experimental import pallas as pl
from jax.experimental.pallas import tpu as pltpu

_B, _D = 16384, 2048
_H = 2048
_BR = 1024
_G = _B // _BR
_WR = _H // _G


def _loss_kernel(yh_ref, y_ref, w1_ref, w2_ref, out_ref, acc_ref):
    i = pl.program_id(0)

    @pl.when(i == 0)
    def _init():
        acc_ref[0] = 0.0
        acc_ref[1] = 0.0
        acc_ref[2] = 0.0

    d = yh_ref[...] - y_ref[...]
    acc_ref[0] += jnp.sum(d * d)
    w1 = w1_ref[...]
    acc_ref[1] += jnp.sum(w1 * w1)
    w2 = w2_ref[...]
    acc_ref[2] += jnp.sum(w2 * w2)

    @pl.when(i == _G - 1)
    def _fin():
        out_ref[0, 0] = (acc_ref[0] / (_B * _D)
                         + jnp.sqrt(acc_ref[1]) + jnp.sqrt(acc_ref[2]))


def kernel(y_hat, y, W1, W2):
    out = pl.pallas_call(
        _loss_kernel,
        grid=(_G,),
        in_specs=[
            pl.BlockSpec((_BR, _D), lambda i: (i, 0)),
            pl.BlockSpec((_BR, _D), lambda i: (i, 0)),
            pl.BlockSpec((_WR, _D), lambda i: (i, 0)),
            pl.BlockSpec((_WR, _H), lambda i: (i, 0)),
        ],
        out_specs=pl.BlockSpec(memory_space=pltpu.SMEM),
        out_shape=jax.ShapeDtypeStruct((1, 1), jnp.float32),
        scratch_shapes=[pltpu.SMEM((3,), jnp.float32)],
        compiler_params=pltpu.CompilerParams(
            dimension_semantics=("arbitrary",),
        ),
    )(y_hat, y, W1, W2)
    return out[0, 0]


# speedup vs baseline: 1.2843x; 1.0120x over previous
import jax
import jax.numpy as jnp
from jax.experimental import pallas as pl
from jax.experimental.pallas import tpu as pltpu

_B, _D = 16384, 2048
_H = 2048
_BR = 512
_G = _B // _BR
_WR = _H // _G


def _loss_kernel(yh_ref, y_ref, w1_ref, w2_ref, out_ref, acc_ref):
    i = pl.program_id(0)

    @pl.when(i == 0)
    def _init():
        acc_ref[0] = 0.0
        acc_ref[1] = 0.0
        acc_ref[2] = 0.0

    d = yh_ref[...] - y_ref[...]
    acc_ref[0] += jnp.sum(d * d)
    w1 = w1_ref[...]
    acc_ref[1] += jnp.sum(w1 * w1)
    w2 = w2_ref[...]
    acc_ref[2] += jnp.sum(w2 * w2)

    @pl.when(i == _G - 1)
    def _fin():
        out_ref[0, 0] = (acc_ref[0] / (_B * _D)
                         + jnp.sqrt(acc_ref[1]) + jnp.sqrt(acc_ref[2]))


def kernel(y_hat, y, W1, W2):
    out = pl.pallas_call(
        _loss_kernel,
        grid=(_G,),
        in_specs=[
            pl.BlockSpec((_BR, _D), lambda i: (i, 0)),
            pl.BlockSpec((_BR, _D), lambda i: (i, 0)),
            pl.BlockSpec((_WR, _D), lambda i: (i, 0)),
            pl.BlockSpec((_WR, _H), lambda i: (i, 0)),
        ],
        out_specs=pl.BlockSpec(memory_space=pltpu.SMEM),
        out_shape=jax.ShapeDtypeStruct((1, 1), jnp.float32),
        scratch_shapes=[pltpu.SMEM((3,), jnp.float32)],
        compiler_params=pltpu.CompilerParams(
            dimension_semantics=("arbitrary",),
        ),
    )(y_hat, y, W1, W2)
    return out[0, 0]
